# baseline (speedup 1.0000x reference)
import functools

import jax
import jax.numpy as jnp
from jax.experimental import pallas as pl
from jax.experimental.pallas import tpu as pltpu

C = 64
G = 8
B = 4096
P = 49
LANES = B * P
CHUNK = P * 128
NSTEPS = LANES // CHUNK
MOM = 0.1
INV_COUNT = 1.0 / (G * B)


def _body(x_ref, m_ref, o_ref, acc_ref):
    j = pl.program_id(0)

    @pl.when(j == 0)
    def _init():
        acc_ref[...] = jnp.zeros_like(acc_ref)

    blk = x_ref[...]
    acc_ref[...] += jnp.sum(blk.reshape(G, G, CHUNK), axis=1)

    @pl.when(j == NSTEPS - 1)
    def _finish():
        i0 = jax.lax.broadcasted_iota(jnp.int32, (CHUNK, P), 0)
        i1 = jax.lax.broadcasted_iota(jnp.int32, (CHUNK, P), 1)
        sel = (i0 % P == i1).astype(jnp.float32)
        s = jax.lax.dot_general(
            acc_ref[...], sel,
            dimension_numbers=(((1,), (0,)), ((), ())),
            preferred_element_type=jnp.float32,
        )
        o_ref[...] = (1.0 - MOM) * m_ref[...] + (MOM * INV_COUNT) * s


@functools.partial(jax.jit)
def kernel(query_attention_maps, group_memory):
    x = query_attention_maps.reshape(C, LANES)
    gm = group_memory.reshape(G, P)
    out = pl.pallas_call(
        _body,
        grid=(NSTEPS,),
        in_specs=[
            pl.BlockSpec((C, CHUNK), lambda j: (0, j)),
            pl.BlockSpec((G, P), lambda j: (0, 0)),
        ],
        out_specs=pl.BlockSpec((G, P), lambda j: (0, 0)),
        out_shape=jax.ShapeDtypeStruct((G, P), jnp.float32),
        scratch_shapes=[pltpu.VMEM((G, CHUNK), jnp.float32)],
    )(x, gm)
    return out.reshape(G, 1, 7, 7)


# speedup vs baseline: 13.0243x; 13.0243x over previous
import jax
import jax.numpy as jnp
from jax.experimental import pallas as pl

C = 64
G = 8
B = 4096
P = 49
PCHUNK = 7
NSTEPS = P // PCHUNK
MOM = 0.1
INV_COUNT = 1.0 / (G * B)


def _body(x_ref, gm_ref, o_ref):
    blk = x_ref[...]
    part = jnp.sum(blk.reshape(PCHUNK, G, G, B), axis=(2, 3))
    o_ref[0] = (1.0 - MOM) * gm_ref[0] + (MOM * INV_COUNT) * part


def kernel(query_attention_maps, group_memory):
    x = jnp.transpose(query_attention_maps, (0, 3, 4, 1, 2)).reshape(P, C, B)
    gm_t = group_memory.reshape(G, P).T.reshape(NSTEPS, PCHUNK, G)
    res = pl.pallas_call(
        _body,
        grid=(NSTEPS,),
        in_specs=[
            pl.BlockSpec((PCHUNK, C, B), lambda j: (j, 0, 0)),
            pl.BlockSpec((1, PCHUNK, G), lambda j: (j, 0, 0)),
        ],
        out_specs=pl.BlockSpec((1, PCHUNK, G), lambda j: (j, 0, 0)),
        out_shape=jax.ShapeDtypeStruct((NSTEPS, PCHUNK, G), jnp.float32),
    )(x, gm_t)
    return res.reshape(P, G).T.reshape(G, 1, 7, 7)
